# baseline (speedup 1.0000x reference)
import functools

import numpy as np
import jax
import jax.numpy as jnp
from jax import lax
from jax.experimental import pallas as pl
from jax.experimental.pallas import tpu as pltpu
from jax.experimental.pallas import tpu_sc as plsc

_L = 16

_CABINP1 = np.array([[-20.0, 3.0], [-10.0, 2.0], [0.0, 0.88], [10.0, 0.38],
                     [25.0, 0.7], [30.0, 1.0], [35.0, 1.31], [40.0, 2.5],
                     [45.0, 3.0]], dtype=np.float64)
_CABINP2 = np.array([[-20.0, 3.0], [-15.0, 2.0], [-10.0, 1.0], [-5.0, 0.5],
                     [0.0, 0.4], [5.0, 0.5], [10.0, 1.5], [15.0, 3.0],
                     [20.0, 6.0]], dtype=np.float64)
_TEMP_SET = np.array([18.0, 20, 22, 24, 26, 28, 30, 31.5, 32], dtype=np.float64)
_TEMP_ENVR = np.array([-30.0, -20, -10, 0, 5, 10, 15, 20, 25, 30, 35, 40, 45, 50],
                      dtype=np.float64)
_CABINSP = np.array([
    [17.0, 17, 17, 17, 17, 17, 17, 17, 17, 17, 17, 17, 17, 17],
    [20, 20, 19.5, 19.5, 19.5, 19, 19, 19, 18.5, 18.5, 18, 18, 18, 18],
    [22, 22, 22, 22.5, 22.5, 22.5, 22, 22, 21, 21, 21, 21, 20.5, 20],
    [24, 24.5, 25.5, 25.5, 26, 26, 25.5, 25, 24.5, 24, 23.5, 23, 23, 23],
    [27, 26.5, 27, 27.5, 28, 28, 27.5, 27, 26.5, 26, 25.5, 26, 26, 26],
    [29, 28.5, 28.5, 29.5, 30, 30, 29.5, 29, 29, 29, 28, 28, 29, 29],
    [31, 30.5, 30.5, 31.5, 32, 32, 32, 31, 31, 31, 31, 31, 31, 31],
    [32, 32, 32, 33, 33, 33, 33, 33, 33, 33, 33, 33, 32, 32],
    [32, 32, 36, 36, 36, 36, 36, 36, 36, 36, 36, 36, 36, 36]], dtype=np.float64)


def _seg_lut(pts, lo, step, n):
    mids = lo + step * (np.arange(n) + 0.5)
    return np.clip(np.searchsorted(pts, mids) - 1, 0, len(pts) - 2).astype(np.int32)


def _line_coefs(xp, fp):
    s = np.diff(fp) / np.diff(xp)
    c = fp[:-1] - s * xp[:-1]
    return c, s

_O_TAB = 0
_O_CA = 126
_O_CB = 139
_O_RA = 152
_O_RB = 160
_O_C1 = 168
_O_S1 = 176
_O_C2 = 184
_O_S2 = 192
_NF = 200

_O_LUTC = 0
_O_LUTR = 17
_O_LUT1 = 46
_NI = 64


def _build_tables():
    tf = np.zeros((_NF,), np.float64)
    tf[_O_TAB:_O_TAB + 126] = _CABINSP.reshape(-1)
    invc = 1.0 / np.diff(_TEMP_ENVR)
    tf[_O_CA:_O_CA + 13] = invc
    tf[_O_CB:_O_CB + 13] = -_TEMP_ENVR[:-1] * invc
    invr = 1.0 / np.diff(_TEMP_SET)
    tf[_O_RA:_O_RA + 8] = invr
    tf[_O_RB:_O_RB + 8] = -_TEMP_SET[:-1] * invr
    c1, s1 = _line_coefs(_CABINP1[:, 0], _CABINP1[:, 1])
    tf[_O_C1:_O_C1 + 8] = c1
    tf[_O_S1:_O_S1 + 8] = s1
    c2, s2 = _line_coefs(_CABINP2[:, 0], _CABINP2[:, 1])
    tf[_O_C2:_O_C2 + 8] = c2
    tf[_O_S2:_O_S2 + 8] = s2

    ti = np.zeros((_NI,), np.int32)
    ti[_O_LUTC:_O_LUTC + 17] = _seg_lut(_TEMP_ENVR, -30.0, 5.0, 17)
    ti[_O_LUTR:_O_LUTR + 29] = _seg_lut(_TEMP_SET, 18.0, 0.5, 29)
    ti[_O_LUT1:_O_LUT1 + 14] = _seg_lut(_CABINP1[:, 0], -20.0, 5.0, 14)
    return tf.astype(np.float32), ti

_TBLF_NP, _TBLI_NP = _build_tables()


def _f(v):
    return np.float32(v)


def _group_compute(tf, ti, a0, fl, fr, tin):
    a = jnp.clip(a0, _f(-30.0), _f(50.0))
    iu = ((a + _f(30.0)) * _f(0.2)).astype(jnp.int32)
    ci0 = plsc.load_gather(ti, [iu])
    tc = a * plsc.load_gather(tf, [ci0 + _O_CA]) \
        + plsc.load_gather(tf, [ci0 + _O_CB])

    def bilin(q):
        qc = jnp.clip(q, _f(18.0), _f(32.0))
        ir = ((qc - _f(18.0)) * _f(2.0)).astype(jnp.int32)
        ri0 = plsc.load_gather(ti, [ir + _O_LUTR])
        tr = qc * plsc.load_gather(tf, [ri0 + _O_RA]) \
            + plsc.load_gather(tf, [ri0 + _O_RB])
        base = ri0 * 14 + ci0
        f00 = plsc.load_gather(tf, [base])
        f01 = plsc.load_gather(tf, [base + 1])
        f10 = plsc.load_gather(tf, [base + 14])
        f11 = plsc.load_gather(tf, [base + 15])
        top = f00 + tc * (f01 - f00)
        bot = f10 + tc * (f11 - f10)
        return top + tr * (bot - top)

    a1 = jnp.clip(a0, _f(-20.0), _f(45.0))
    i1 = ((a1 + _f(20.0)) * _f(0.2)).astype(jnp.int32)
    s1 = plsc.load_gather(ti, [i1 + _O_LUT1])
    kp1 = plsc.load_gather(tf, [s1 + _O_C1]) \
        + plsc.load_gather(tf, [s1 + _O_S1]) * a1

    err = jnp.minimum(bilin(fl), bilin(fr)) - tin
    e = jnp.clip(err, _f(-20.0), _f(20.0))
    i2 = jnp.minimum(((e + _f(20.0)) * _f(0.2)).astype(jnp.int32), 7)
    kp2 = plsc.load_gather(tf, [i2 + _O_C2]) \
        + plsc.load_gather(tf, [i2 + _O_S2]) * e

    return jnp.minimum(kp1, kp2)


_COLS = (1, 2, 3, 8)


@functools.cache
def _make_sc_kernel(nrows):
    NW = 32
    R = nrows // NW
    CH = 8192
    NCH = R // CH
    NPAIR = NCH // 2
    CR = CH // 128
    GRP = CH // _L
    mesh = plsc.VectorSubcoreMesh(core_axis_name="c", subcore_axis_name="s")

    col_scratch = [pltpu.VMEM((CR, 128), jnp.float32)] * (2 * len(_COLS))

    @functools.partial(
        pl.kernel, mesh=mesh,
        compiler_params=pltpu.CompilerParams(needs_layout_passes=False,
                                             use_tc_tiling_on_sc=False),
        out_type=jax.ShapeDtypeStruct((nrows // 128, 128), jnp.float32),
        scratch_types=col_scratch + [
            pltpu.VMEM((CR, 128), jnp.float32),
            pltpu.VMEM((CR, 128), jnp.float32),
            pltpu.VMEM((_NF,), jnp.float32),
            pltpu.VMEM((_NI,), jnp.int32),
            pltpu.SemaphoreType.DMA,
            pltpu.SemaphoreType.DMA,
        ],
    )
    def sc_kernel(w_hbm, tf_hbm, ti_hbm, out_hbm,
                  a0, f0, g0, t0, a1, f1, g1, t1,
                  ov0, ov1, tfv, tiv, si0, si1):
        wid = lax.axis_index("s") * 2 + lax.axis_index("c")
        base_blk = wid * (R // 128)

        pltpu.sync_copy(tf_hbm, tfv)
        pltpu.sync_copy(ti_hbm, tiv)

        bufs = ((a0, f0, g0, t0), (a1, f1, g1, t1))

        def start_in(ch, bufset, sem):
            blk = base_blk + ch * CR
            for col, dst in zip(_COLS, bufset):
                pltpu.make_async_copy(
                    w_hbm.at[col // 8, pl.ds(blk, CR), col % 8, :],
                    dst, sem).start()

        def wait_in(bufset, sem):
            for dst in bufset:
                pltpu.make_async_copy(
                    w_hbm.at[0, pl.ds(0, CR), 0, :], dst, sem).wait()

        def compute(bufset, ov):
            av, fv, gv, tv = bufset

            @plsc.parallel_loop(0, GRP, unroll=1)
            def body(g):
                rr = g // 8
                cc = (g % 8) * _L
                a = av[rr, pl.ds(cc, _L)]
                fl = fv[rr, pl.ds(cc, _L)]
                fr = gv[rr, pl.ds(cc, _L)]
                ti = tv[rr, pl.ds(cc, _L)]
                ov[rr, pl.ds(cc, _L)] = _group_compute(tfv, tiv, a, fl, fr, ti)

        start_in(0, bufs[0], si0)
        start_in(1, bufs[1], si1)

        def pair(p, carry):
            chA = p * 2
            wait_in(bufs[0], si0)
            compute(bufs[0], ov0)

            @pl.when(p < NPAIR - 1)
            def _():
                start_in(chA + 2, bufs[0], si0)

            pltpu.sync_copy(ov0, out_hbm.at[pl.ds(base_blk + chA * CR, CR), :])

            wait_in(bufs[1], si1)
            compute(bufs[1], ov1)

            @pl.when(p < NPAIR - 1)
            def _():
                start_in(chA + 3, bufs[1], si1)

            pltpu.sync_copy(
                ov1, out_hbm.at[pl.ds(base_blk + (chA + 1) * CR, CR), :])
            return carry

        lax.fori_loop(0, NPAIR, pair, 0)

    return sc_kernel


def kernel(x):
    if x.ndim == 1:
        x = x[None, :]
    nrows, ncols = x.shape
    w = x.reshape(nrows // 128, 128, ncols // 8, 8).transpose(2, 0, 3, 1)
    out = _make_sc_kernel(nrows)(w, jnp.asarray(_TBLF_NP), jnp.asarray(_TBLI_NP))
    return out.reshape(nrows, 1)


# speedup vs baseline: 1.0218x; 1.0218x over previous
import functools

import numpy as np
import jax
import jax.numpy as jnp
from jax import lax
from jax.experimental import pallas as pl
from jax.experimental.pallas import tpu as pltpu
from jax.experimental.pallas import tpu_sc as plsc

_L = 16

_CABINP1 = np.array([[-20.0, 3.0], [-10.0, 2.0], [0.0, 0.88], [10.0, 0.38],
                     [25.0, 0.7], [30.0, 1.0], [35.0, 1.31], [40.0, 2.5],
                     [45.0, 3.0]], dtype=np.float64)
_CABINP2 = np.array([[-20.0, 3.0], [-15.0, 2.0], [-10.0, 1.0], [-5.0, 0.5],
                     [0.0, 0.4], [5.0, 0.5], [10.0, 1.5], [15.0, 3.0],
                     [20.0, 6.0]], dtype=np.float64)
_TEMP_SET = np.array([18.0, 20, 22, 24, 26, 28, 30, 31.5, 32], dtype=np.float64)
_TEMP_ENVR = np.array([-30.0, -20, -10, 0, 5, 10, 15, 20, 25, 30, 35, 40, 45, 50],
                      dtype=np.float64)
_CABINSP = np.array([
    [17.0, 17, 17, 17, 17, 17, 17, 17, 17, 17, 17, 17, 17, 17],
    [20, 20, 19.5, 19.5, 19.5, 19, 19, 19, 18.5, 18.5, 18, 18, 18, 18],
    [22, 22, 22, 22.5, 22.5, 22.5, 22, 22, 21, 21, 21, 21, 20.5, 20],
    [24, 24.5, 25.5, 25.5, 26, 26, 25.5, 25, 24.5, 24, 23.5, 23, 23, 23],
    [27, 26.5, 27, 27.5, 28, 28, 27.5, 27, 26.5, 26, 25.5, 26, 26, 26],
    [29, 28.5, 28.5, 29.5, 30, 30, 29.5, 29, 29, 29, 28, 28, 29, 29],
    [31, 30.5, 30.5, 31.5, 32, 32, 32, 31, 31, 31, 31, 31, 31, 31],
    [32, 32, 32, 33, 33, 33, 33, 33, 33, 33, 33, 33, 32, 32],
    [32, 32, 36, 36, 36, 36, 36, 36, 36, 36, 36, 36, 36, 36]], dtype=np.float64)


def _seg_lut(pts, lo, step, n):
    mids = lo + step * (np.arange(n) + 0.5)
    return np.clip(np.searchsorted(pts, mids) - 1, 0, len(pts) - 2).astype(np.int32)


def _line_coefs(xp, fp):
    s = np.diff(fp) / np.diff(xp)
    c = fp[:-1] - s * xp[:-1]
    return c, s

_O_TAB = 0
_O_CA = 126
_O_CB = 139
_O_RA = 152
_O_RB = 160
_O_C1 = 168
_O_S1 = 176
_O_C2 = 184
_O_S2 = 192
_NF = 200

_O_LUTC = 0
_O_LUTR = 17
_O_LUT1 = 46
_NI = 64


def _build_tables():
    tf = np.zeros((_NF,), np.float64)
    tf[_O_TAB:_O_TAB + 126] = _CABINSP.reshape(-1)
    invc = 1.0 / np.diff(_TEMP_ENVR)
    tf[_O_CA:_O_CA + 13] = invc
    tf[_O_CB:_O_CB + 13] = -_TEMP_ENVR[:-1] * invc
    invr = 1.0 / np.diff(_TEMP_SET)
    tf[_O_RA:_O_RA + 8] = invr
    tf[_O_RB:_O_RB + 8] = -_TEMP_SET[:-1] * invr
    c1, s1 = _line_coefs(_CABINP1[:, 0], _CABINP1[:, 1])
    tf[_O_C1:_O_C1 + 8] = c1
    tf[_O_S1:_O_S1 + 8] = s1
    c2, s2 = _line_coefs(_CABINP2[:, 0], _CABINP2[:, 1])
    tf[_O_C2:_O_C2 + 8] = c2
    tf[_O_S2:_O_S2 + 8] = s2

    ti = np.zeros((_NI,), np.int32)
    ti[_O_LUTC:_O_LUTC + 17] = _seg_lut(_TEMP_ENVR, -30.0, 5.0, 17)
    ti[_O_LUTR:_O_LUTR + 29] = _seg_lut(_TEMP_SET, 18.0, 0.5, 29)
    ti[_O_LUT1:_O_LUT1 + 14] = _seg_lut(_CABINP1[:, 0], -20.0, 5.0, 14)
    return tf.astype(np.float32), ti

_TBLF_NP, _TBLI_NP = _build_tables()


def _f(v):
    return np.float32(v)


def _group_compute(tf, ti, a0, fl, fr, tin):
    a = jnp.clip(a0, _f(-30.0), _f(50.0))
    iu = ((a + _f(30.0)) * _f(0.2)).astype(jnp.int32)
    ci0 = plsc.load_gather(ti, [iu])
    tc = a * plsc.load_gather(tf, [ci0 + _O_CA]) \
        + plsc.load_gather(tf, [ci0 + _O_CB])

    def bilin(q):
        qc = jnp.clip(q, _f(18.0), _f(32.0))
        ir = ((qc - _f(18.0)) * _f(2.0)).astype(jnp.int32)
        ri0 = plsc.load_gather(ti, [ir + _O_LUTR])
        tr = qc * plsc.load_gather(tf, [ri0 + _O_RA]) \
            + plsc.load_gather(tf, [ri0 + _O_RB])
        base = ri0 * 14 + ci0
        f00 = plsc.load_gather(tf, [base])
        f01 = plsc.load_gather(tf, [base + 1])
        f10 = plsc.load_gather(tf, [base + 14])
        f11 = plsc.load_gather(tf, [base + 15])
        top = f00 + tc * (f01 - f00)
        bot = f10 + tc * (f11 - f10)
        return top + tr * (bot - top)

    a1 = jnp.clip(a0, _f(-20.0), _f(45.0))
    i1 = ((a1 + _f(20.0)) * _f(0.2)).astype(jnp.int32)
    s1 = plsc.load_gather(ti, [i1 + _O_LUT1])
    kp1 = plsc.load_gather(tf, [s1 + _O_C1]) \
        + plsc.load_gather(tf, [s1 + _O_S1]) * a1

    err = jnp.minimum(bilin(fl), bilin(fr)) - tin
    e = jnp.clip(err, _f(-20.0), _f(20.0))
    i2 = jnp.minimum(((e + _f(20.0)) * _f(0.2)).astype(jnp.int32), 7)
    kp2 = plsc.load_gather(tf, [i2 + _O_C2]) \
        + plsc.load_gather(tf, [i2 + _O_S2]) * e

    return jnp.minimum(kp1, kp2)


_COLS = (1, 2, 3, 8)


@functools.cache
def _make_sc_kernel(nrows):
    NW = 32
    R = nrows // NW
    CH = 2048
    NCH = R // CH
    NPAIR = NCH // 2
    CR = CH // 128
    GRP = CH // _L
    mesh = plsc.VectorSubcoreMesh(core_axis_name="c", subcore_axis_name="s")

    col_scratch = [pltpu.VMEM((CR, 128), jnp.float32)] * (2 * len(_COLS))

    @functools.partial(
        pl.kernel, mesh=mesh,
        compiler_params=pltpu.CompilerParams(needs_layout_passes=False,
                                             use_tc_tiling_on_sc=False),
        out_type=jax.ShapeDtypeStruct((nrows // 128, 128), jnp.float32),
        scratch_types=col_scratch + [
            pltpu.VMEM((CR, 128), jnp.float32),
            pltpu.VMEM((CR, 128), jnp.float32),
            pltpu.VMEM((_NF,), jnp.float32),
            pltpu.VMEM((_NI,), jnp.int32),
            pltpu.SemaphoreType.DMA,
            pltpu.SemaphoreType.DMA,
        ],
    )
    def sc_kernel(w_hbm, tf_hbm, ti_hbm, out_hbm,
                  a0, f0, g0, t0, a1, f1, g1, t1,
                  ov0, ov1, tfv, tiv, si0, si1):
        wid = lax.axis_index("s") * 2 + lax.axis_index("c")
        base_blk = wid * (R // 128)

        pltpu.sync_copy(tf_hbm, tfv)
        pltpu.sync_copy(ti_hbm, tiv)

        bufs = ((a0, f0, g0, t0), (a1, f1, g1, t1))

        def start_in(ch, bufset, sem):
            blk = base_blk + ch * CR
            for col, dst in zip(_COLS, bufset):
                pltpu.make_async_copy(
                    w_hbm.at[col // 8, pl.ds(blk, CR), col % 8, :],
                    dst, sem).start()

        def wait_in(bufset, sem):
            for dst in bufset:
                pltpu.make_async_copy(
                    w_hbm.at[0, pl.ds(0, CR), 0, :], dst, sem).wait()

        def compute(bufset, ov):
            av, fv, gv, tv = bufset

            @plsc.parallel_loop(0, GRP, unroll=1)
            def body(g):
                rr = g // 8
                cc = (g % 8) * _L
                a = av[rr, pl.ds(cc, _L)]
                fl = fv[rr, pl.ds(cc, _L)]
                fr = gv[rr, pl.ds(cc, _L)]
                ti = tv[rr, pl.ds(cc, _L)]
                ov[rr, pl.ds(cc, _L)] = _group_compute(tfv, tiv, a, fl, fr, ti)

        start_in(0, bufs[0], si0)
        start_in(1, bufs[1], si1)

        def pair(p, carry):
            chA = p * 2
            wait_in(bufs[0], si0)
            compute(bufs[0], ov0)

            @pl.when(p < NPAIR - 1)
            def _():
                start_in(chA + 2, bufs[0], si0)

            pltpu.sync_copy(ov0, out_hbm.at[pl.ds(base_blk + chA * CR, CR), :])

            wait_in(bufs[1], si1)
            compute(bufs[1], ov1)

            @pl.when(p < NPAIR - 1)
            def _():
                start_in(chA + 3, bufs[1], si1)

            pltpu.sync_copy(
                ov1, out_hbm.at[pl.ds(base_blk + (chA + 1) * CR, CR), :])
            return carry

        lax.fori_loop(0, NPAIR, pair, 0)

    return sc_kernel


def kernel(x):
    if x.ndim == 1:
        x = x[None, :]
    nrows, ncols = x.shape
    w = x.reshape(nrows // 128, 128, ncols // 8, 8).transpose(2, 0, 3, 1)
    out = _make_sc_kernel(nrows)(w, jnp.asarray(_TBLF_NP), jnp.asarray(_TBLI_NP))
    return out.reshape(nrows, 1)
